# baseline (speedup 1.0000x reference)
import functools

import numpy as np
import jax
import jax.numpy as jnp
from jax import lax
from jax.experimental import pallas as pl
from jax.experimental.pallas import tpu as pltpu
from jax.experimental.pallas import tpu_sc as plsc

_F32 = jnp.float32
_CSRC = 128

_SC_CORES = 2
_SC_SUBCORES = 16
_SC_WORKERS = _SC_CORES * _SC_SUBCORES


def _elu(x):
    return jnp.where(x > 0, x, jnp.exp(jnp.minimum(x, 0.0)) - 1.0)


def _knn_body(kds, N, Npad, Ppad, *refs):
    pts_ref, rept_ref = refs[0], refs[1]
    idx_refs = refs[2:]

    b = pl.program_id(0)
    pts = pts_ref[0]
    rep_t = rept_ref[0]

    G = jnp.dot(pts, rep_t, preferred_element_type=_F32)
    rq = jnp.sum(rep_t * rep_t, axis=0)
    rp = jnp.sum(pts * pts, axis=1, keepdims=True)
    distT = (rq[None, :] - 2.0 * G) + rp

    riota = lax.broadcasted_iota(jnp.int32, (Npad, Ppad), 0)
    if Npad > N:
        distT = jnp.where(riota >= N, jnp.inf, distT)

    selmap = {}
    rounds = 0
    for li, (K, D) in enumerate(kds):
        for j in range(K):
            selmap.setdefault(1 + j * D, []).append((li, j))
        rounds = max(rounds, (K - 1) * D + 2)

    m = jnp.min(distT, axis=0)
    for r in range(rounds):
        t = jnp.where(distT <= m[None, :], riota, Npad)
        idx = jnp.min(t, axis=0)
        for li, j in selmap.get(r, []):
            idx_refs[li][0, j, :] = idx + b * Npad
        if r + 1 < rounds:
            msk = riota == idx[None, :]
            distT = jnp.where(msk, jnp.inf, distT)
            m = jnp.min(distT, axis=0)


def _sc_gather(table, idx):
    V, C = table.shape
    R = idx.shape[0]
    per_w = R // _SC_WORKERS
    ch = 512 if per_w % 512 == 0 else per_w
    nch = per_w // ch
    mesh = plsc.VectorSubcoreMesh(core_axis_name="c", subcore_axis_name="s")

    @functools.partial(
        pl.kernel, mesh=mesh,
        out_type=jax.ShapeDtypeStruct((R, C), _F32),
        scratch_types=[
            pltpu.VMEM((ch,), jnp.int32),
            pltpu.VMEM((ch, C), _F32),
            pltpu.SemaphoreType.DMA,
        ],
    )
    def k(table_hbm, idx_hbm, out_hbm, idx_v, rows_v, sem):
        wid = lax.axis_index("s") * _SC_CORES + lax.axis_index("c")
        base = wid * per_w

        def body(i, carry):
            off = base + i * ch
            pltpu.sync_copy(idx_hbm.at[pl.ds(off, ch)], idx_v)
            pltpu.async_copy(table_hbm.at[idx_v], rows_v, sem).wait()
            pltpu.sync_copy(rows_v, out_hbm.at[pl.ds(off, ch)])
            return carry

        lax.fori_loop(0, nch, body, 0)

    return k(table, idx)


def _xconv_body(K, Cin, C2, Cm, dm, Ct, Cout, Ppad, Nn, Cn2, *refs):
    if Cn2 > 0:
        (rep_ref, gath_ref, d1W, d1b, d2W, d2b,
         xcW2, xcb, xd1W, xd1b, xd2W, xd2b, dwW2, pwW, pwb,
         inWn, inbn, out_ref) = refs
    else:
        (rep_ref, gath_ref, d1W, d1b, d2W, d2b,
         xcW2, xcb, xd1W, xd1b, xd2W, xd2b, dwW2, pwW, pwb, out_ref) = refs

    rep = rep_ref[0]
    gath = gath_ref[0]
    gsel = [gath[k * Ppad:(k + 1) * Ppad] for k in range(K)]

    pls = [g[:, :3] - rep for g in gsel]
    PL = jnp.concatenate(pls, axis=0)

    f = _elu(jnp.dot(PL, d1W[...], preferred_element_type=_F32) + d1b[...])
    f = _elu(jnp.dot(f, d2W[...], preferred_element_type=_F32) + d2b[...])

    acc = None
    for k in range(K):
        term = jnp.dot(pls[k], xcW2[k], preferred_element_type=_F32)
        acc = term if acc is None else acc + term
    Xc = _elu(acc + xcb[...])
    X1 = _elu(jnp.dot(Xc, xd1W[...], preferred_element_type=_F32) + xd1b[...])
    X = jnp.dot(X1, xd2W[...], preferred_element_type=_F32) + xd2b[...]

    zpad = jnp.zeros((Ppad, _CSRC - 3 - C2 - Cm), _F32)
    fcat = [jnp.concatenate([gsel[j][:, :3 + C2],
                             f[j * Ppad:(j + 1) * Ppad], zpad], axis=1)
            for j in range(K)]

    dws = [jnp.zeros((Ppad, _CSRC), _F32) for _ in range(dm)]
    for i in range(K):
        fX = None
        for j in range(K):
            xij = X[:, i * K + j][:, None]
            t = xij * fcat[j]
            fX = t if fX is None else fX + t
        for mi in range(dm):
            dws[mi] = dws[mi] + fX * dwW2[i * dm + mi, :][None, :]
    dwcat = jnp.concatenate(dws, axis=1)
    out = _elu(jnp.dot(dwcat, pwW[...], preferred_element_type=_F32) + pwb[...])

    if Cn2 > 0:
        nxt = _elu(jnp.dot(out, inWn[...], preferred_element_type=_F32)
                   + inbn[...])
        pad = _CSRC - 3 - Cn2
        out_ref[0] = jnp.concatenate(
            [rep, nxt, jnp.zeros((Ppad, pad), _F32)], axis=1)
    else:
        out_ref[0] = out


def _batch_spec(B):
    def spec(a):
        if a.ndim == 3 and a.shape[0] == B:
            return pl.BlockSpec((1,) + a.shape[1:], lambda b: (b, 0, 0))
        nd = a.ndim
        return pl.BlockSpec(a.shape, lambda b, _nd=nd: (0,) * _nd)
    return spec


def _run_knn(pts, rep_t, kds, N):
    B, Npad = pts.shape[0], pts.shape[1]
    Ppad = rep_t.shape[2]
    spec = _batch_spec(B)
    body = functools.partial(_knn_body, kds, N, Npad, Ppad)
    out = pl.pallas_call(
        body,
        grid=(B,),
        in_specs=[spec(pts), spec(rep_t)],
        out_specs=[pl.BlockSpec((1, K, Ppad), lambda b: (b, 0, 0))
                   for K, _ in kds],
        out_shape=[jax.ShapeDtypeStruct((B, K, Ppad), jnp.int32)
                   for K, _ in kds],
        compiler_params=pltpu.CompilerParams(
            dimension_semantics=("parallel",)),
    )(pts, rep_t)
    return out if isinstance(out, (list, tuple)) else [out]


def _run_xconv(rep, gath, p, Cin, Cout, K, nxt):
    B, Ppad = rep.shape[0], rep.shape[1]
    C2 = Cout // 2 if Cin > 0 else 0
    Cm = Cout // 2 if Cin == 0 else Cout // 4
    dm = 1 if Cin == 0 else min(int(np.ceil(Cout / Cin)), 4)
    Ct = Cm + C2
    spec = _batch_spec(B)

    xcW2 = jnp.transpose(p["xc_W"], (2, 1, 0))
    dwp = jnp.transpose(p["dw_W"], (2, 1, 0)).reshape(K * dm, Ct)
    dwW2 = jnp.zeros((K * dm, _CSRC), _F32)
    if C2 > 0:
        dwW2 = dwW2.at[:, 3:3 + C2].set(dwp[:, Cm:Ct])
    dwW2 = dwW2.at[:, 3 + C2:3 + C2 + Cm].set(dwp[:, :Cm])
    pwp = p["pw_W"].reshape(Ct, dm, Cout).transpose(1, 0, 2)
    pwW = jnp.zeros((dm, _CSRC, Cout), _F32)
    if C2 > 0:
        pwW = pwW.at[:, 3:3 + C2, :].set(pwp[:, Cm:Ct, :])
    pwW = pwW.at[:, 3 + C2:3 + C2 + Cm, :].set(pwp[:, :Cm, :])
    pwW = pwW.reshape(dm * _CSRC, Cout)
    row = lambda v: v.reshape(1, -1)

    ins = [rep, gath,
           p["d1_W"], row(p["d1_b"]), p["d2_W"], row(p["d2_b"]),
           xcW2, row(p["xc_b"]), p["xd1_W"], row(p["xd1_b"]),
           p["xd2_W"], row(p["xd2_b"]), dwW2, pwW, row(p["pw_b"])]
    if nxt is not None:
        Cn2 = nxt[0].shape[1]
        ins += [nxt[0], row(nxt[1])]
        out_w = _CSRC
    else:
        Cn2 = 0
        out_w = Cout
    body = functools.partial(_xconv_body, K, Cin, C2, Cm, dm, Ct, Cout,
                             Ppad, 0, Cn2)
    return pl.pallas_call(
        body,
        grid=(B,),
        in_specs=[spec(a) for a in ins],
        out_specs=pl.BlockSpec((1, Ppad, out_w), lambda b: (b, 0, 0)),
        out_shape=jax.ShapeDtypeStruct((B, Ppad, out_w), _F32),
        compiler_params=pltpu.CompilerParams(
            dimension_semantics=("parallel",)),
    )(*ins)


def kernel(x, params):
    B, N0 = x.shape[0], x.shape[1]
    cfg = {"l1": (0, 32, 8, 1), "l2": (32, 64, 8, 2), "l3": (64, 96, 8, 4),
           "l4": (96, 128, 12, 4), "l5": (128, 30, 12, 6)}

    i4 = np.sort(np.random.default_rng(3).choice(N0, size=120, replace=False))
    rep4 = x[:, i4, :]
    rep4 = jnp.pad(rep4, ((0, 0), (0, 8), (0, 0)))
    rep4_t = jnp.transpose(rep4, (0, 2, 1))
    x_t = jnp.transpose(x, (0, 2, 1))

    idx1, idx2, idx3 = _run_knn(x, x_t, [(8, 1), (8, 2), (8, 4)], N0)
    (idx4,) = _run_knn(x, rep4_t, [(12, 4)], N0)
    (idx5,) = _run_knn(rep4, rep4_t, [(12, 6)], 120)

    def gath(src, idx, K, Ppad):
        g = _sc_gather(src.reshape(-1, _CSRC), idx.reshape(-1))
        return g.reshape(B, K * Ppad, _CSRC)

    src1 = jnp.pad(x, ((0, 0), (0, 0), (0, _CSRC - 3)))
    pr = params
    src2 = _run_xconv(x, gath(src1, idx1, 8, N0), pr["l1"], 0, 32, 8,
                      (pr["l2"]["in_W"], pr["l2"]["in_b"]))
    src3 = _run_xconv(x, gath(src2, idx2, 8, N0), pr["l2"], 32, 64, 8,
                      (pr["l3"]["in_W"], pr["l3"]["in_b"]))
    src4 = _run_xconv(x, gath(src3, idx3, 8, N0), pr["l3"], 64, 96, 8,
                      (pr["l4"]["in_W"], pr["l4"]["in_b"]))
    src5 = _run_xconv(rep4, gath(src4, idx4, 12, 128), pr["l4"], 96, 128, 12,
                      (pr["l5"]["in_W"], pr["l5"]["in_b"]))
    fts = _run_xconv(rep4, gath(src5, idx5, 12, 128), pr["l5"], 128, 30, 12,
                     None)
    return fts[:, :120, :].reshape(B, 3, 1200)


# speedup vs baseline: 1.0100x; 1.0100x over previous
import functools

import numpy as np
import jax
import jax.numpy as jnp
from jax import lax
from jax.experimental import pallas as pl
from jax.experimental.pallas import tpu as pltpu
from jax.experimental.pallas import tpu_sc as plsc

_F32 = jnp.float32
_CSRC = 128

_SC_CORES = 2
_SC_SUBCORES = 16
_SC_WORKERS = _SC_CORES * _SC_SUBCORES


def _elu(x):
    return jnp.where(x > 0, x, jnp.exp(jnp.minimum(x, 0.0)) - 1.0)


def _knn_body(kds, N, Npad, Ppad, *refs):
    pts_ref, rept_ref = refs[0], refs[1]
    idx_refs = refs[2:]

    b = pl.program_id(0)
    pts = pts_ref[0]
    rep_t = rept_ref[0]

    G = jnp.dot(pts, rep_t, preferred_element_type=_F32)
    rq = jnp.sum(rep_t * rep_t, axis=0)
    rp = jnp.sum(pts * pts, axis=1, keepdims=True)
    distT = (rq[None, :] - 2.0 * G) + rp

    riota = lax.broadcasted_iota(jnp.int32, (Npad, Ppad), 0)
    if Npad > N:
        distT = jnp.where(riota >= N, jnp.inf, distT)

    selmap = {}
    rounds = 0
    for li, (K, D) in enumerate(kds):
        for j in range(K):
            selmap.setdefault(1 + j * D, []).append((li, j))
        rounds = max(rounds, (K - 1) * D + 2)

    m = jnp.min(distT, axis=0)
    for r in range(rounds):
        t = jnp.where(distT <= m[None, :], riota, Npad)
        idx = jnp.min(t, axis=0)
        for li, j in selmap.get(r, []):
            idx_refs[li][0, j, :] = idx + b * Npad
        if r + 1 < rounds:
            msk = riota == idx[None, :]
            distT = jnp.where(msk, jnp.inf, distT)
            m = jnp.min(distT, axis=0)


def _sc_gather(table, idx):
    V, C = table.shape
    R = idx.shape[0]
    per_w = R // _SC_WORKERS
    ch = 256 if per_w % 256 == 0 else per_w
    nch = per_w // ch
    mesh = plsc.VectorSubcoreMesh(core_axis_name="c", subcore_axis_name="s")

    @functools.partial(
        pl.kernel, mesh=mesh,
        out_type=jax.ShapeDtypeStruct((R, C), _F32),
        scratch_types=[
            pltpu.VMEM((ch,), jnp.int32),
            pltpu.VMEM((ch,), jnp.int32),
            pltpu.VMEM((ch, C), _F32),
            pltpu.VMEM((ch, C), _F32),
            pltpu.SemaphoreType.DMA,
            pltpu.SemaphoreType.DMA,
        ],
    )
    def k(table_hbm, idx_hbm, out_hbm, i0, i1, r0, r1, s0, s1):
        wid = lax.axis_index("s") * _SC_CORES + lax.axis_index("c")
        base = wid * per_w
        ibufs, rbufs, sems = [i0, i1], [r0, r1], [s0, s1]
        handles = [None, None]

        pltpu.sync_copy(idx_hbm.at[pl.ds(base, ch)], i0)
        handles[0] = pltpu.async_copy(table_hbm.at[i0], r0, s0)
        for i in range(nch):
            cur, nxt = i % 2, (i + 1) % 2
            if i + 1 < nch:
                off = base + (i + 1) * ch
                pltpu.sync_copy(idx_hbm.at[pl.ds(off, ch)], ibufs[nxt])
                handles[nxt] = pltpu.async_copy(table_hbm.at[ibufs[nxt]],
                                                rbufs[nxt], sems[nxt])
            handles[cur].wait()
            pltpu.sync_copy(rbufs[cur], out_hbm.at[pl.ds(base + i * ch, ch)])

    return k(table, idx)


def _xconv_body(K, Cin, C2, Cm, dm, Ct, Cout, Ppad, Nn, Cn2, *refs):
    if Cn2 > 0:
        (rep_ref, gath_ref, d1W, d1b, d2W, d2b,
         xcW2, xcb, xd1W, xd1b, xd2W, xd2b, dwW2, pwW, pwb,
         inWn, inbn, out_ref) = refs
    else:
        (rep_ref, gath_ref, d1W, d1b, d2W, d2b,
         xcW2, xcb, xd1W, xd1b, xd2W, xd2b, dwW2, pwW, pwb, out_ref) = refs

    rep = rep_ref[0]
    gath = gath_ref[0]
    gsel = [gath[k * Ppad:(k + 1) * Ppad] for k in range(K)]

    pls = [g[:, :3] - rep for g in gsel]
    PL = jnp.concatenate(pls, axis=0)

    f = _elu(jnp.dot(PL, d1W[...], preferred_element_type=_F32) + d1b[...])
    f = _elu(jnp.dot(f, d2W[...], preferred_element_type=_F32) + d2b[...])

    acc = None
    for k in range(K):
        term = jnp.dot(pls[k], xcW2[k], preferred_element_type=_F32)
        acc = term if acc is None else acc + term
    Xc = _elu(acc + xcb[...])
    X1 = _elu(jnp.dot(Xc, xd1W[...], preferred_element_type=_F32) + xd1b[...])
    X = jnp.dot(X1, xd2W[...], preferred_element_type=_F32) + xd2b[...]

    dwf = [jnp.zeros((Ppad, Cm), _F32) for _ in range(dm)]
    dwg = [jnp.zeros((Ppad, C2), _F32) for _ in range(dm)] if Cin > 0 else None
    for i in range(K):
        fXf = None
        fXg = None
        for j in range(K):
            xij = X[:, i * K + j][:, None]
            tf = xij * f[j * Ppad:(j + 1) * Ppad]
            fXf = tf if fXf is None else fXf + tf
            if Cin > 0:
                tg = xij * gsel[j][:, 3:3 + C2]
                fXg = tg if fXg is None else fXg + tg
        for mi in range(dm):
            w = dwW2[i * dm + mi, :][None, :]
            dwf[mi] = dwf[mi] + fXf * w[:, :Cm]
            if Cin > 0:
                dwg[mi] = dwg[mi] + fXg * w[:, Cm:]

    parts = []
    for mi in range(dm):
        parts.append(dwf[mi])
        if Cin > 0:
            parts.append(dwg[mi])
    dwcat = jnp.concatenate(parts, axis=1)
    out = _elu(jnp.dot(dwcat, pwW[...], preferred_element_type=_F32) + pwb[...])

    if Cn2 > 0:
        nxt = _elu(jnp.dot(out, inWn[...], preferred_element_type=_F32)
                   + inbn[...])
        pad = _CSRC - 3 - Cn2
        out_ref[0] = jnp.concatenate(
            [rep, nxt, jnp.zeros((Ppad, pad), _F32)], axis=1)
    else:
        out_ref[0] = out


def _batch_spec(B):
    def spec(a):
        if a.ndim == 3 and a.shape[0] == B:
            return pl.BlockSpec((1,) + a.shape[1:], lambda b: (b, 0, 0))
        nd = a.ndim
        return pl.BlockSpec(a.shape, lambda b, _nd=nd: (0,) * _nd)
    return spec


def _run_knn(pts, rep_t, kds, N):
    B, Npad = pts.shape[0], pts.shape[1]
    Ppad = rep_t.shape[2]
    spec = _batch_spec(B)
    body = functools.partial(_knn_body, kds, N, Npad, Ppad)
    out = pl.pallas_call(
        body,
        grid=(B,),
        in_specs=[spec(pts), spec(rep_t)],
        out_specs=[pl.BlockSpec((1, K, Ppad), lambda b: (b, 0, 0))
                   for K, _ in kds],
        out_shape=[jax.ShapeDtypeStruct((B, K, Ppad), jnp.int32)
                   for K, _ in kds],
        compiler_params=pltpu.CompilerParams(
            dimension_semantics=("parallel",)),
    )(pts, rep_t)
    return out if isinstance(out, (list, tuple)) else [out]


def _run_xconv(rep, gath, p, Cin, Cout, K, nxt):
    B, Ppad = rep.shape[0], rep.shape[1]
    C2 = Cout // 2 if Cin > 0 else 0
    Cm = Cout // 2 if Cin == 0 else Cout // 4
    dm = 1 if Cin == 0 else min(int(np.ceil(Cout / Cin)), 4)
    Ct = Cm + C2
    spec = _batch_spec(B)

    xcW2 = jnp.transpose(p["xc_W"], (2, 1, 0))
    dwW2 = jnp.transpose(p["dw_W"], (2, 1, 0)).reshape(K * dm, Ct)
    pwW = (p["pw_W"].reshape(Ct, dm, Cout)
           .transpose(1, 0, 2).reshape(dm * Ct, Cout))
    row = lambda v: v.reshape(1, -1)

    ins = [rep, gath,
           p["d1_W"], row(p["d1_b"]), p["d2_W"], row(p["d2_b"]),
           xcW2, row(p["xc_b"]), p["xd1_W"], row(p["xd1_b"]),
           p["xd2_W"], row(p["xd2_b"]), dwW2, pwW, row(p["pw_b"])]
    if nxt is not None:
        Cn2 = nxt[0].shape[1]
        ins += [nxt[0], row(nxt[1])]
        out_w = _CSRC
    else:
        Cn2 = 0
        out_w = Cout
    body = functools.partial(_xconv_body, K, Cin, C2, Cm, dm, Ct, Cout,
                             Ppad, 0, Cn2)
    return pl.pallas_call(
        body,
        grid=(B,),
        in_specs=[spec(a) for a in ins],
        out_specs=pl.BlockSpec((1, Ppad, out_w), lambda b: (b, 0, 0)),
        out_shape=jax.ShapeDtypeStruct((B, Ppad, out_w), _F32),
        compiler_params=pltpu.CompilerParams(
            dimension_semantics=("parallel",)),
    )(*ins)


def kernel(x, params):
    B, N0 = x.shape[0], x.shape[1]
    cfg = {"l1": (0, 32, 8, 1), "l2": (32, 64, 8, 2), "l3": (64, 96, 8, 4),
           "l4": (96, 128, 12, 4), "l5": (128, 30, 12, 6)}

    i4 = np.sort(np.random.default_rng(3).choice(N0, size=120, replace=False))
    rep4 = x[:, i4, :]
    rep4 = jnp.pad(rep4, ((0, 0), (0, 8), (0, 0)))
    rep4_t = jnp.transpose(rep4, (0, 2, 1))
    x_t = jnp.transpose(x, (0, 2, 1))

    idx1, idx2, idx3 = _run_knn(x, x_t, [(8, 1), (8, 2), (8, 4)], N0)
    (idx4,) = _run_knn(x, rep4_t, [(12, 4)], N0)
    (idx5,) = _run_knn(rep4, rep4_t, [(12, 6)], 120)

    def gath(src, idx, K, Ppad):
        g = _sc_gather(src.reshape(-1, _CSRC), idx.reshape(-1))
        return g.reshape(B, K * Ppad, _CSRC)

    src1 = jnp.pad(x, ((0, 0), (0, 0), (0, _CSRC - 3)))
    pr = params
    src2 = _run_xconv(x, gath(src1, idx1, 8, N0), pr["l1"], 0, 32, 8,
                      (pr["l2"]["in_W"], pr["l2"]["in_b"]))
    src3 = _run_xconv(x, gath(src2, idx2, 8, N0), pr["l2"], 32, 64, 8,
                      (pr["l3"]["in_W"], pr["l3"]["in_b"]))
    src4 = _run_xconv(x, gath(src3, idx3, 8, N0), pr["l3"], 64, 96, 8,
                      (pr["l4"]["in_W"], pr["l4"]["in_b"]))
    src5 = _run_xconv(rep4, gath(src4, idx4, 12, 128), pr["l4"], 96, 128, 12,
                      (pr["l5"]["in_W"], pr["l5"]["in_b"]))
    fts = _run_xconv(rep4, gath(src5, idx5, 12, 128), pr["l5"], 128, 30, 12,
                     None)
    return fts[:, :120, :].reshape(B, 3, 1200)


# speedup vs baseline: 1.0227x; 1.0126x over previous
import functools

import numpy as np
import jax
import jax.numpy as jnp
from jax import lax
from jax.experimental import pallas as pl
from jax.experimental.pallas import tpu as pltpu
from jax.experimental.pallas import tpu_sc as plsc

_F32 = jnp.float32
_CSRC = 128

_SC_CORES = 2
_SC_SUBCORES = 16
_SC_WORKERS = _SC_CORES * _SC_SUBCORES


def _elu(x):
    return jnp.where(x > 0, x, jnp.exp(jnp.minimum(x, 0.0)) - 1.0)


def _knn_body(kds, N, Npad, Ppad, *refs):
    pts_ref, rept_ref = refs[0], refs[1]
    idx_refs = refs[2:]

    b = pl.program_id(0)
    pts = pts_ref[0]
    rep_t = rept_ref[0]

    G = jnp.dot(pts, rep_t, preferred_element_type=_F32)
    rq = jnp.sum(rep_t * rep_t, axis=0)
    rp = jnp.sum(pts * pts, axis=1, keepdims=True)
    distT = (rq[None, :] - 2.0 * G) + rp

    riota = lax.broadcasted_iota(jnp.int32, (Npad, Ppad), 0)
    if Npad > N:
        distT = jnp.where(riota >= N, jnp.inf, distT)

    selmap = {}
    rounds = 0
    for li, (K, D) in enumerate(kds):
        for j in range(K):
            selmap.setdefault(1 + j * D, []).append((li, j))
        rounds = max(rounds, (K - 1) * D + 2)

    m = jnp.min(distT, axis=0)
    for r in range(rounds):
        t = jnp.where(distT <= m[None, :], riota, Npad)
        idx = jnp.min(t, axis=0)
        for li, j in selmap.get(r, []):
            idx_refs[li][0, j, :] = idx + b * Npad
        if r + 1 < rounds:
            msk = riota == idx[None, :]
            distT = jnp.where(msk, jnp.inf, distT)
            m = jnp.min(distT, axis=0)


def _sc_gather(table, idx):
    V, C = table.shape
    R = idx.shape[0]
    per_w = R // _SC_WORKERS
    ch = 256 if per_w % 256 == 0 else per_w
    nch = per_w // ch
    mesh = plsc.VectorSubcoreMesh(core_axis_name="c", subcore_axis_name="s")

    @functools.partial(
        pl.kernel, mesh=mesh,
        out_type=jax.ShapeDtypeStruct((R, C), _F32),
        scratch_types=[
            pltpu.VMEM((ch,), jnp.int32),
            pltpu.VMEM((ch,), jnp.int32),
            pltpu.VMEM((ch, C), _F32),
            pltpu.VMEM((ch, C), _F32),
            pltpu.SemaphoreType.DMA,
            pltpu.SemaphoreType.DMA,
        ],
    )
    def k(table_hbm, idx_hbm, out_hbm, i0, i1, r0, r1, s0, s1):
        wid = lax.axis_index("s") * _SC_CORES + lax.axis_index("c")
        base = wid * per_w
        ibufs, rbufs, sems = [i0, i1], [r0, r1], [s0, s1]
        handles = [None, None]

        pltpu.sync_copy(idx_hbm.at[pl.ds(base, ch)], i0)
        handles[0] = pltpu.async_copy(table_hbm.at[i0], r0, s0)
        for i in range(nch):
            cur, nxt = i % 2, (i + 1) % 2
            if i + 1 < nch:
                off = base + (i + 1) * ch
                pltpu.sync_copy(idx_hbm.at[pl.ds(off, ch)], ibufs[nxt])
                handles[nxt] = pltpu.async_copy(table_hbm.at[ibufs[nxt]],
                                                rbufs[nxt], sems[nxt])
            handles[cur].wait()
            pltpu.sync_copy(rbufs[cur], out_hbm.at[pl.ds(base + i * ch, ch)])

    return k(table, idx)


def _xconv_body(K, Cin, C2, Cm, dm, Ct, Cout, Ppad, Nn, Cn2, *refs):
    if Cn2 > 0:
        (rep_ref, gath_ref, d1W, d1b, d2W, d2b,
         xcW2, xcb, xd1W, xd1b, xd2W, xd2b, dwW2, pwW, pwb,
         inWn, inbn, out_ref) = refs
    else:
        (rep_ref, gath_ref, d1W, d1b, d2W, d2b,
         xcW2, xcb, xd1W, xd1b, xd2W, xd2b, dwW2, pwW, pwb, out_ref) = refs

    rep = rep_ref[0]
    gath = gath_ref[0]
    gsel = [gath[k * Ppad:(k + 1) * Ppad] for k in range(K)]

    pls = [g[:, :3] - rep for g in gsel]
    PL = jnp.concatenate(pls, axis=0)

    f = _elu(jnp.dot(PL, d1W[...], preferred_element_type=_F32) + d1b[...])
    f = _elu(jnp.dot(f, d2W[...], preferred_element_type=_F32) + d2b[...])

    PLW = jnp.concatenate(pls, axis=1)
    acc = jnp.dot(PLW, xcW2[...], preferred_element_type=_F32)
    Xc = _elu(acc + xcb[...])
    X1 = _elu(jnp.dot(Xc, xd1W[...], preferred_element_type=_F32) + xd1b[...])
    X = jnp.dot(X1, xd2W[...], preferred_element_type=_F32) + xd2b[...]

    dwf = [jnp.zeros((Ppad, Cm), _F32) for _ in range(dm)]
    dwg = [jnp.zeros((Ppad, C2), _F32) for _ in range(dm)] if Cin > 0 else None
    for i in range(K):
        fXf = None
        fXg = None
        for j in range(K):
            xij = X[:, i * K + j][:, None]
            tf = xij * f[j * Ppad:(j + 1) * Ppad]
            fXf = tf if fXf is None else fXf + tf
            if Cin > 0:
                tg = xij * gsel[j][:, 3:3 + C2]
                fXg = tg if fXg is None else fXg + tg
        for mi in range(dm):
            w = dwW2[i * dm + mi, :][None, :]
            dwf[mi] = dwf[mi] + fXf * w[:, :Cm]
            if Cin > 0:
                dwg[mi] = dwg[mi] + fXg * w[:, Cm:]

    parts = []
    for mi in range(dm):
        parts.append(dwf[mi])
        if Cin > 0:
            parts.append(dwg[mi])
    dwcat = jnp.concatenate(parts, axis=1)
    out = _elu(jnp.dot(dwcat, pwW[...], preferred_element_type=_F32) + pwb[...])

    if Cn2 > 0:
        nxt = _elu(jnp.dot(out, inWn[...], preferred_element_type=_F32)
                   + inbn[...])
        pad = _CSRC - 3 - Cn2
        out_ref[0] = jnp.concatenate(
            [rep, nxt, jnp.zeros((Ppad, pad), _F32)], axis=1)
    else:
        out_ref[0] = out


def _batch_spec(B):
    def spec(a):
        if a.ndim == 3 and a.shape[0] == B:
            return pl.BlockSpec((1,) + a.shape[1:], lambda b: (b, 0, 0))
        nd = a.ndim
        return pl.BlockSpec(a.shape, lambda b, _nd=nd: (0,) * _nd)
    return spec


def _run_knn(pts, rep_t, kds, N):
    B, Npad = pts.shape[0], pts.shape[1]
    Ppad = rep_t.shape[2]
    spec = _batch_spec(B)
    body = functools.partial(_knn_body, kds, N, Npad, Ppad)
    out = pl.pallas_call(
        body,
        grid=(B,),
        in_specs=[spec(pts), spec(rep_t)],
        out_specs=[pl.BlockSpec((1, K, Ppad), lambda b: (b, 0, 0))
                   for K, _ in kds],
        out_shape=[jax.ShapeDtypeStruct((B, K, Ppad), jnp.int32)
                   for K, _ in kds],
        compiler_params=pltpu.CompilerParams(
            dimension_semantics=("parallel",)),
    )(pts, rep_t)
    return out if isinstance(out, (list, tuple)) else [out]


def _run_xconv(rep, gath, p, Cin, Cout, K, nxt):
    B, Ppad = rep.shape[0], rep.shape[1]
    C2 = Cout // 2 if Cin > 0 else 0
    Cm = Cout // 2 if Cin == 0 else Cout // 4
    dm = 1 if Cin == 0 else min(int(np.ceil(Cout / Cin)), 4)
    Ct = Cm + C2
    spec = _batch_spec(B)

    xcW2 = jnp.transpose(p["xc_W"], (2, 1, 0)).reshape(3 * K, K * K)
    dwW2 = jnp.transpose(p["dw_W"], (2, 1, 0)).reshape(K * dm, Ct)
    pwW = (p["pw_W"].reshape(Ct, dm, Cout)
           .transpose(1, 0, 2).reshape(dm * Ct, Cout))
    row = lambda v: v.reshape(1, -1)

    ins = [rep, gath,
           p["d1_W"], row(p["d1_b"]), p["d2_W"], row(p["d2_b"]),
           xcW2, row(p["xc_b"]), p["xd1_W"], row(p["xd1_b"]),
           p["xd2_W"], row(p["xd2_b"]), dwW2, pwW, row(p["pw_b"])]
    if nxt is not None:
        Cn2 = nxt[0].shape[1]
        ins += [nxt[0], row(nxt[1])]
        out_w = _CSRC
    else:
        Cn2 = 0
        out_w = Cout
    body = functools.partial(_xconv_body, K, Cin, C2, Cm, dm, Ct, Cout,
                             Ppad, 0, Cn2)
    return pl.pallas_call(
        body,
        grid=(B,),
        in_specs=[spec(a) for a in ins],
        out_specs=pl.BlockSpec((1, Ppad, out_w), lambda b: (b, 0, 0)),
        out_shape=jax.ShapeDtypeStruct((B, Ppad, out_w), _F32),
        compiler_params=pltpu.CompilerParams(
            dimension_semantics=("parallel",)),
    )(*ins)


def kernel(x, params):
    B, N0 = x.shape[0], x.shape[1]
    cfg = {"l1": (0, 32, 8, 1), "l2": (32, 64, 8, 2), "l3": (64, 96, 8, 4),
           "l4": (96, 128, 12, 4), "l5": (128, 30, 12, 6)}

    i4 = np.sort(np.random.default_rng(3).choice(N0, size=120, replace=False))
    rep4 = x[:, i4, :]
    rep4 = jnp.pad(rep4, ((0, 0), (0, 8), (0, 0)))
    rep4_t = jnp.transpose(rep4, (0, 2, 1))
    x_t = jnp.transpose(x, (0, 2, 1))

    idx1, idx2, idx3 = _run_knn(x, x_t, [(8, 1), (8, 2), (8, 4)], N0)
    (idx4,) = _run_knn(x, rep4_t, [(12, 4)], N0)
    (idx5,) = _run_knn(rep4, rep4_t, [(12, 6)], 120)

    def gath(src, idx, K, Ppad):
        g = _sc_gather(src.reshape(-1, _CSRC), idx.reshape(-1))
        return g.reshape(B, K * Ppad, _CSRC)

    src1 = jnp.pad(x, ((0, 0), (0, 0), (0, _CSRC - 3)))
    pr = params
    src2 = _run_xconv(x, gath(src1, idx1, 8, N0), pr["l1"], 0, 32, 8,
                      (pr["l2"]["in_W"], pr["l2"]["in_b"]))
    src3 = _run_xconv(x, gath(src2, idx2, 8, N0), pr["l2"], 32, 64, 8,
                      (pr["l3"]["in_W"], pr["l3"]["in_b"]))
    src4 = _run_xconv(x, gath(src3, idx3, 8, N0), pr["l3"], 64, 96, 8,
                      (pr["l4"]["in_W"], pr["l4"]["in_b"]))
    src5 = _run_xconv(rep4, gath(src4, idx4, 12, 128), pr["l4"], 96, 128, 12,
                      (pr["l5"]["in_W"], pr["l5"]["in_b"]))
    fts = _run_xconv(rep4, gath(src5, idx5, 12, 128), pr["l5"], 128, 30, 12,
                     None)
    return fts[:, :120, :].reshape(B, 3, 1200)


# speedup vs baseline: 1.0229x; 1.0002x over previous
import functools

import numpy as np
import jax
import jax.numpy as jnp
from jax import lax
from jax.experimental import pallas as pl
from jax.experimental.pallas import tpu as pltpu
from jax.experimental.pallas import tpu_sc as plsc

_F32 = jnp.float32
_CSRC = 128

_SC_CORES = 2
_SC_SUBCORES = 16
_SC_WORKERS = _SC_CORES * _SC_SUBCORES


def _elu(x):
    return jnp.where(x > 0, x, jnp.exp(jnp.minimum(x, 0.0)) - 1.0)


def _knn_body(kds, N, Npad, Ppad, *refs):
    pts_ref, rept_ref = refs[0], refs[1]
    idx_refs = refs[2:]

    b = pl.program_id(0)
    pts = pts_ref[0]
    rep_t = rept_ref[0]

    G = jnp.dot(pts, rep_t, preferred_element_type=_F32)
    rq = jnp.sum(rep_t * rep_t, axis=0)
    rp = jnp.sum(pts * pts, axis=1, keepdims=True)
    distT = (rq[None, :] - 2.0 * G) + rp

    riota = lax.broadcasted_iota(jnp.int32, (Npad, Ppad), 0)
    if Npad > N:
        distT = jnp.where(riota >= N, jnp.inf, distT)

    selmap = {}
    rounds = 0
    for li, (K, D) in enumerate(kds):
        for j in range(K):
            selmap.setdefault(1 + j * D, []).append((li, j))
        rounds = max(rounds, (K - 1) * D + 2)

    m = jnp.min(distT, axis=0)
    for r in range(rounds):
        t = jnp.where(distT <= m[None, :], riota, Npad)
        idx = jnp.min(t, axis=0)
        for li, j in selmap.get(r, []):
            idx_refs[li][0, j, :] = idx + b * Npad
        if r + 1 < rounds:
            msk = riota == idx[None, :]
            distT = jnp.where(msk, jnp.inf, distT)
            m = jnp.min(distT, axis=0)


def _sc_gather(table, idx):
    V, C = table.shape
    R = idx.shape[0]
    per_w = R // _SC_WORKERS
    ch = 256 if per_w % 256 == 0 else per_w
    nch = per_w // ch
    mesh = plsc.VectorSubcoreMesh(core_axis_name="c", subcore_axis_name="s")

    @functools.partial(
        pl.kernel, mesh=mesh,
        out_type=jax.ShapeDtypeStruct((R, C), _F32),
        scratch_types=[
            pltpu.VMEM((ch,), jnp.int32),
            pltpu.VMEM((ch,), jnp.int32),
            pltpu.VMEM((ch, C), _F32),
            pltpu.VMEM((ch, C), _F32),
            pltpu.SemaphoreType.DMA,
            pltpu.SemaphoreType.DMA,
        ],
    )
    def k(table_hbm, idx_hbm, out_hbm, i0, i1, r0, r1, s0, s1):
        wid = lax.axis_index("s") * _SC_CORES + lax.axis_index("c")
        base = wid * per_w
        ibufs, rbufs, sems = [i0, i1], [r0, r1], [s0, s1]
        handles = [None, None]

        pltpu.sync_copy(idx_hbm.at[pl.ds(base, ch)], i0)
        handles[0] = pltpu.async_copy(table_hbm.at[i0], r0, s0)
        for i in range(nch):
            cur, nxt = i % 2, (i + 1) % 2
            if i + 1 < nch:
                off = base + (i + 1) * ch
                pltpu.sync_copy(idx_hbm.at[pl.ds(off, ch)], ibufs[nxt])
                handles[nxt] = pltpu.async_copy(table_hbm.at[ibufs[nxt]],
                                                rbufs[nxt], sems[nxt])
            handles[cur].wait()
            pltpu.sync_copy(rbufs[cur], out_hbm.at[pl.ds(base + i * ch, ch)])

    return k(table, idx)


def _xconv_body(K, Cin, C2, Cm, dm, Ct, Cout, Ppad, Nn, Cn2, *refs):
    if Cn2 > 0:
        (rep_ref, gath_ref, d1W, d1b, d2W, d2b,
         xcW2, xcb, xd1W, xd1b, xd2W, xd2b, dwW2, pwW, pwb,
         inWn, inbn, out_ref) = refs
    else:
        (rep_ref, gath_ref, d1W, d1b, d2W, d2b,
         xcW2, xcb, xd1W, xd1b, xd2W, xd2b, dwW2, pwW, pwb, out_ref) = refs

    rep = rep_ref[0]
    gath = gath_ref[0]
    gsel = [gath[k * Ppad:(k + 1) * Ppad] for k in range(K)]

    pls = [g[:, :3] - rep for g in gsel]
    PL = jnp.concatenate(pls, axis=0)

    f = _elu(jnp.dot(PL, d1W[...], preferred_element_type=_F32) + d1b[...])
    f = _elu(jnp.dot(f, d2W[...], preferred_element_type=_F32) + d2b[...])

    PLW = jnp.concatenate(pls, axis=1)
    acc = jnp.dot(PLW, xcW2[...], preferred_element_type=_F32)
    Xc = _elu(acc + xcb[...])
    X1 = _elu(jnp.dot(Xc, xd1W[...], preferred_element_type=_F32) + xd1b[...])
    X = jnp.dot(X1, xd2W[...], preferred_element_type=_F32) + xd2b[...]

    dwf = [jnp.zeros((Ppad, Cm), _F32) for _ in range(dm)]
    dwg = [jnp.zeros((Ppad, C2), _F32) for _ in range(dm)] if Cin > 0 else None
    for i in range(K):
        fXf = None
        fXg = None
        for j in range(K):
            xij = X[:, i * K + j][:, None]
            tf = xij * f[j * Ppad:(j + 1) * Ppad]
            fXf = tf if fXf is None else fXf + tf
            if Cin > 0:
                tg = xij * gsel[j][:, 3:3 + C2]
                fXg = tg if fXg is None else fXg + tg
        for mi in range(dm):
            w = dwW2[i * dm + mi, :][None, :]
            dwf[mi] = dwf[mi] + fXf * w[:, :Cm]
            if Cin > 0:
                dwg[mi] = dwg[mi] + fXg * w[:, Cm:]

    parts = []
    for mi in range(dm):
        parts.append(dwf[mi])
        if Cin > 0:
            parts.append(dwg[mi])
    dwcat = jnp.concatenate(parts, axis=1)
    out = _elu(jnp.dot(dwcat, pwW[...], preferred_element_type=_F32) + pwb[...])

    if Cn2 > 0:
        nxt = _elu(jnp.dot(out, inWn[...], preferred_element_type=_F32)
                   + inbn[...])
        pad = _CSRC - 3 - Cn2
        out_ref[0] = jnp.concatenate(
            [rep, nxt, jnp.zeros((Ppad, pad), _F32)], axis=1)
    else:
        out_ref[0] = out


def _batch_spec(B):
    def spec(a):
        if a.ndim == 3 and a.shape[0] == B:
            return pl.BlockSpec((1,) + a.shape[1:], lambda b: (b, 0, 0))
        nd = a.ndim
        return pl.BlockSpec(a.shape, lambda b, _nd=nd: (0,) * _nd)
    return spec


def _run_knn(pts, rep_t, kds, N):
    B, Npad = pts.shape[0], pts.shape[1]
    Ppad = rep_t.shape[2]
    spec = _batch_spec(B)
    body = functools.partial(_knn_body, kds, N, Npad, Ppad)
    out = pl.pallas_call(
        body,
        grid=(B,),
        in_specs=[spec(pts), spec(rep_t)],
        out_specs=[pl.BlockSpec((1, K, Ppad), lambda b: (b, 0, 0))
                   for K, _ in kds],
        out_shape=[jax.ShapeDtypeStruct((B, K, Ppad), jnp.int32)
                   for K, _ in kds],
        compiler_params=pltpu.CompilerParams(
            dimension_semantics=("parallel",)),
    )(pts, rep_t)
    return out if isinstance(out, (list, tuple)) else [out]


def _run_xconv(rep, gath, p, Cin, Cout, K, nxt):
    B, Ppad = rep.shape[0], rep.shape[1]
    C2 = Cout // 2 if Cin > 0 else 0
    Cm = Cout // 2 if Cin == 0 else Cout // 4
    dm = 1 if Cin == 0 else min(int(np.ceil(Cout / Cin)), 4)
    Ct = Cm + C2
    spec = _batch_spec(B)

    xcW2 = jnp.transpose(p["xc_W"], (2, 1, 0)).reshape(3 * K, K * K)
    dwW2 = jnp.transpose(p["dw_W"], (2, 1, 0)).reshape(K * dm, Ct)
    pwW = (p["pw_W"].reshape(Ct, dm, Cout)
           .transpose(1, 0, 2).reshape(dm * Ct, Cout))
    row = lambda v: v.reshape(1, -1)

    ins = [rep, gath,
           p["d1_W"], row(p["d1_b"]), p["d2_W"], row(p["d2_b"]),
           xcW2, row(p["xc_b"]), p["xd1_W"], row(p["xd1_b"]),
           p["xd2_W"], row(p["xd2_b"]), dwW2, pwW, row(p["pw_b"])]
    if nxt is not None:
        Cn2 = nxt[0].shape[1]
        ins += [nxt[0], row(nxt[1])]
        out_w = _CSRC
    else:
        Cn2 = 0
        out_w = Cout
    body = functools.partial(_xconv_body, K, Cin, C2, Cm, dm, Ct, Cout,
                             Ppad, 0, Cn2)
    return pl.pallas_call(
        body,
        grid=(B,),
        in_specs=[spec(a) for a in ins],
        out_specs=pl.BlockSpec((1, Ppad, out_w), lambda b: (b, 0, 0)),
        out_shape=jax.ShapeDtypeStruct((B, Ppad, out_w), _F32),
        compiler_params=pltpu.CompilerParams(
            dimension_semantics=("parallel",)),
    )(*ins)


def kernel(x, params):
    B, N0 = x.shape[0], x.shape[1]

    i4 = np.sort(np.random.default_rng(3).choice(N0, size=120, replace=False))
    rep4 = x[:, i4, :]
    rep4 = jnp.pad(rep4, ((0, 0), (0, 8), (0, 0)))
    rep4_t = jnp.transpose(rep4, (0, 2, 1))
    x_t = jnp.transpose(x, (0, 2, 1))

    idx1, idx2, idx3 = _run_knn(x, x_t, [(8, 1), (8, 2), (8, 4)], N0)
    (idx4,) = _run_knn(x, rep4_t, [(12, 4)], N0)
    (idx5,) = _run_knn(rep4, rep4_t, [(12, 6)], 120)

    def gath(src, idx, K, Ppad):
        g = _sc_gather(src.reshape(-1, _CSRC), idx.reshape(-1))
        return g.reshape(B, K * Ppad, _CSRC)

    src1 = jnp.pad(x, ((0, 0), (0, 0), (0, _CSRC - 3)))
    pr = params
    src2 = _run_xconv(x, gath(src1, idx1, 8, N0), pr["l1"], 0, 32, 8,
                      (pr["l2"]["in_W"], pr["l2"]["in_b"]))
    src3 = _run_xconv(x, gath(src2, idx2, 8, N0), pr["l2"], 32, 64, 8,
                      (pr["l3"]["in_W"], pr["l3"]["in_b"]))
    src4 = _run_xconv(x, gath(src3, idx3, 8, N0), pr["l3"], 64, 96, 8,
                      (pr["l4"]["in_W"], pr["l4"]["in_b"]))
    src5 = _run_xconv(rep4, gath(src4, idx4, 12, 128), pr["l4"], 96, 128, 12,
                      (pr["l5"]["in_W"], pr["l5"]["in_b"]))
    fts = _run_xconv(rep4, gath(src5, idx5, 12, 128), pr["l5"], 128, 30, 12,
                     None)
    return fts[:, :120, :].reshape(B, 3, 1200)
